# baseline (device time: 84753 ns/iter reference)
import jax
import jax.numpy as jnp
from jax import lax
from jax.experimental import pallas as pl
from jax.experimental.pallas import tpu as pltpu

N_DEV = 8
M_PER = 512
K = 4096

S_OFF = (0, 192, 352)
S_ROWS = (192, 160, 160)
MASKS = ((1, 3, 4), (3, 4, 1), (4, 1, 3))
OFFS = tuple(
    ([0], [0, m[0]], [0, m[0], m[1], m[0] ^ m[1]]) for m in MASKS
)


def kernel(x, w_mat, scale_x, scale_w):
    m_per, k = x.shape
    _, n_per = w_mat.shape
    assert (m_per, k) == (M_PER, K)

    dot_dims = (((1,), (0,)), ((), ()))

    def body(x_hbm, w_hbm, sx_ref, sw_ref, out_hbm, comm_ref, xf_ref,
             wf_ref, w8_ref, ob_ref, local_sems, out_sems,
             send_sems, recv_sems, recv3_sems):
        my = lax.axis_index("i")

        cp_x = []
        for s in range(3):
            cp = pltpu.make_async_copy(
                x_hbm.at[pl.ds(S_OFF[s], S_ROWS[s]), :],
                xf_ref.at[pl.ds(S_OFF[s], S_ROWS[s]), :],
                local_sems.at[s],
            )
            cp.start()
            cp_x.append(cp)

        barrier_sem = pltpu.get_barrier_semaphore()
        for mask in (1, 3, 4):
            pl.semaphore_signal(
                barrier_sem, inc=1,
                device_id=(my ^ mask,), device_id_type=pl.DeviceIdType.MESH,
            )
        pl.semaphore_wait(barrier_sem, 3)

        scale = sx_ref[0] * sw_ref[0]

        def stripe_ref(off, s):
            return comm_ref.at[off, pl.ds(S_OFF[s], S_ROWS[s]), :]

        def make_rdma(s, r, off, j=None):
            m = MASKS[s][r]
            recv_sem = (
                recv3_sems.at[s, j] if r == 2 else recv_sems.at[s, r]
            )
            return pltpu.make_async_remote_copy(
                src_ref=stripe_ref(off, s),
                dst_ref=stripe_ref(off ^ m, s),
                send_sem=send_sems.at[s, r],
                recv_sem=recv_sem,
                device_id=(my ^ m,),
                device_id_type=pl.DeviceIdType.MESH,
            )

        ob_last = [0, 0]
        ob_slot = [0]

        def ob_wait(slot):
            if ob_last[slot]:
                pltpu.make_async_copy(
                    ob_ref.at[slot, pl.ds(0, ob_last[slot]), :],
                    out_hbm.at[pl.ds(0, ob_last[slot]), :],
                    out_sems.at[slot],
                ).wait()
                ob_last[slot] = 0

        def emit_rows(acc, row0, rows):
            slot = ob_slot[0]
            ob_slot[0] ^= 1
            ob_wait(slot)
            ob_ref[slot, pl.ds(0, rows), :] = jnp.maximum(acc * scale, 0.0)
            pltpu.make_async_copy(
                ob_ref.at[slot, pl.ds(0, rows), :],
                out_hbm.at[pl.ds(row0, rows), :],
                out_sems.at[slot],
            ).start()
            ob_last[slot] = rows

        def stripe_gemm(s, off):
            acc = lax.dot_general(
                stripe_ref(off, s)[...], w8_ref[...], dot_dims,
                preferred_element_type=jnp.float32,
            )
            emit_rows(acc, (my ^ off) * M_PER + S_OFF[s], S_ROWS[s])

        r0 = []
        for s in range(3):
            cp_x[s].wait()
            comm_ref[0, pl.ds(S_OFF[s], S_ROWS[s]), :] = xf_ref[
                pl.ds(S_OFF[s], S_ROWS[s]), :
            ].astype(jnp.float8_e4m3fn)
            rdma = make_rdma(s, 0, 0)
            rdma.start()
            r0.append(rdma)

        for half in range(2):
            cp_w = pltpu.make_async_copy(
                w_hbm.at[pl.ds(half * (K // 2), K // 2), :],
                wf_ref, local_sems.at[3 + half],
            )
            cp_w.start()
            cp_w.wait()
            w8_ref[pl.ds(half * (K // 2), K // 2), :] = wf_ref[...].astype(
                jnp.float8_e5m2
            )

        r1 = [[] for _ in range(3)]
        for s in (1, 2, 0):
            r0[s].wait()
            for off in OFFS[s][1]:
                rdma = make_rdma(s, 1, off)
                rdma.start()
                r1[s].append(rdma)

        acc = lax.dot_general(
            comm_ref[0], w8_ref[...], dot_dims,
            preferred_element_type=jnp.float32,
        )
        emit_rows(acc, my * M_PER, M_PER)
        for s in range(3):
            stripe_gemm(s, MASKS[s][0])

        r2 = [[None] * 4 for _ in range(3)]
        for s in (1, 2, 0):
            for rdma in r1[s]:
                rdma.wait()
            for j, off in enumerate(OFFS[s][2]):
                rdma = make_rdma(s, 2, off, j=j)
                rdma.start()
                r2[s][j] = rdma
        for s in range(3):
            m0, m1 = MASKS[s][0], MASKS[s][1]
            stripe_gemm(s, m1)
            stripe_gemm(s, m0 ^ m1)
        for j in range(4):
            for s in (1, 2, 0):
                r2[s][j].wait()
                stripe_gemm(s, MASKS[s][2] ^ OFFS[s][2][j])

        for slot in range(2):
            ob_wait(slot)

    return pl.pallas_call(
        body,
        out_shape=jax.ShapeDtypeStruct((N_DEV * m_per, n_per), jnp.float32),
        in_specs=[
            pl.BlockSpec(memory_space=pl.ANY),
            pl.BlockSpec(memory_space=pl.ANY),
            pl.BlockSpec(memory_space=pltpu.SMEM),
            pl.BlockSpec(memory_space=pltpu.SMEM),
        ],
        out_specs=pl.BlockSpec(memory_space=pl.ANY),
        scratch_shapes=[
            pltpu.VMEM((N_DEV, M_PER, K), jnp.float8_e4m3fn),
            pltpu.VMEM((M_PER, K), jnp.float32),
            pltpu.VMEM((K // 2, n_per), jnp.float32),
            pltpu.VMEM((K, n_per), jnp.float8_e5m2),
            pltpu.VMEM((2, M_PER, n_per), jnp.float32),
            pltpu.SemaphoreType.DMA((5,)),
            pltpu.SemaphoreType.DMA((2,)),
            pltpu.SemaphoreType.DMA((3, 3)),
            pltpu.SemaphoreType.DMA((3, 2)),
            pltpu.SemaphoreType.DMA((3, 4)),
        ],
        compiler_params=pltpu.CompilerParams(
            collective_id=0,
            vmem_limit_bytes=100 * 1024 * 1024,
        ),
    )(x, w_mat, scale_x, scale_w)
